# Initial kernel scaffold; baseline (speedup 1.0000x reference)
import jax
import jax.numpy as jnp
from jax.experimental import pallas as pl


def kernel(weight, activations):
    raise NotImplementedError("write your pallas kernel here")


# speedup vs baseline: 89.3008x; 89.3008x over previous
import functools

import jax
import jax.numpy as jnp
from jax.experimental import pallas as pl

_CREATE_THRESHOLD = 0.8
_PRUNE_THRESHOLD = 0.01
_MAX_SYNAPSES = 100
_MIN_SYNAPSES = 10
_INIT_STRENGTH = 0.01

_N = 4096
_S = 512
_RPB = 256


def _plasticity_block(w_ref, act_ref, actc_ref, out_ref,
                      created_ref, pruned_ref, spars_ref):
    i = pl.program_id(0)

    @pl.when(i == 0)
    def _init():
        created_ref[...] = jnp.zeros_like(created_ref)
        pruned_ref[...] = jnp.zeros_like(pruned_ref)
        spars_ref[...] = jnp.zeros_like(spars_ref)

    w = w_ref[...]
    absw = jnp.abs(w)
    cnt = jnp.sum((absw > 1e-10).astype(jnp.int32), axis=1, keepdims=True)
    has_room = cnt < _MAX_SYNAPSES

    def _prune_and_tally(wc, created_add):
        a = jnp.abs(wc)
        weak = a < _PRUNE_THRESHOLD
        count2 = jnp.sum((a >= _PRUNE_THRESHOLD).astype(jnp.int32),
                         axis=1, keepdims=True)
        pm = weak & (count2 > _MIN_SYNAPSES)
        w2 = jnp.where(pm, 0.0, wc)
        out_ref[...] = w2
        created_ref[...] += created_add.reshape(1, 1)
        pruned_ref[...] += jnp.sum(pm.astype(jnp.int32)).reshape(1, 1)
        spars_ref[...] += jnp.sum(
            (jnp.abs(w2) < 1e-10).astype(jnp.float32)).reshape(1, 1)

    @pl.when(jnp.any(has_room))
    def _rare_create_path():
        def _norm(x):
            mu = jnp.mean(x, axis=0, keepdims=True)
            c = x - mu
            var = jnp.sum(c * c, axis=0, keepdims=True) / (_S - 1)
            return c / jnp.maximum(jnp.sqrt(var), 1e-8)

        nfull = _norm(act_ref[...])
        ncol = _norm(actc_ref[...])
        corr = jax.lax.dot_general(
            ncol, nfull, (((0,), (0,)), ((), ())),
            preferred_element_type=jnp.float32) * (1.0 / _S)

        abscorr = jnp.abs(corr)
        cand = (abscorr > _CREATE_THRESHOLD) & (absw < 1e-10)
        cand_cnt = jnp.sum(cand.astype(jnp.int32), axis=1, keepdims=True)
        room = jnp.maximum(_MAX_SYNAPSES - cnt, 0)
        n = jnp.where(has_room, jnp.minimum(cand_cnt, room), 0)

        m = jnp.where(cand, abscorr, 0.0)
        mb = jax.lax.bitcast_convert_type(m, jnp.int32)

        def _bis_val(_, carry):
            lo, hi = carry
            mid = lo + (hi - lo) // 2
            c_ge = jnp.sum((mb >= mid).astype(jnp.int32), axis=1,
                           keepdims=True)
            ge = c_ge >= n
            return jnp.where(ge, mid, lo), jnp.where(ge, hi, mid)

        lo0 = jnp.zeros((_RPB, 1), jnp.int32)
        hi0 = jnp.full((_RPB, 1), 0x7F800000, jnp.int32)
        t, _ = jax.lax.fori_loop(0, 31, _bis_val, (lo0, hi0))

        strict = jnp.sum((mb > t).astype(jnp.int32), axis=1, keepdims=True)
        need_eq = n - strict
        eq = (mb == t) & cand
        col = jax.lax.broadcasted_iota(jnp.int32, (_RPB, _N), 1)

        def _bis_col(_, carry):
            lo2, hi2 = carry
            mid = lo2 + (hi2 - lo2) // 2
            c_le = jnp.sum((eq & (col <= mid)).astype(jnp.int32), axis=1,
                           keepdims=True)
            ok = c_le >= need_eq
            return jnp.where(ok, lo2, mid), jnp.where(ok, mid, hi2)

        lo2 = jnp.full((_RPB, 1), -1, jnp.int32)
        hi2 = jnp.full((_RPB, 1), _N - 1, jnp.int32)
        _, cidx = jax.lax.fori_loop(0, 13, _bis_col, (lo2, hi2))

        create = ((mb > t) | (eq & (col <= cidx))) & cand & (n > 0)
        wc = jnp.where(create, _INIT_STRENGTH * jnp.sign(corr), w)
        _prune_and_tally(wc, jnp.sum(n))

    @pl.when(jnp.logical_not(jnp.any(has_room)))
    def _common_path():
        _prune_and_tally(w, jnp.int32(0))

    @pl.when(i == pl.num_programs(0) - 1)
    def _finish():
        spars_ref[...] = spars_ref[...] * (1.0 / (_N * _N))


@functools.partial(jax.jit, static_argnames=())
def kernel(weight, activations):
    grid = (_N // _RPB,)
    w_out, created, pruned, spars = pl.pallas_call(
        _plasticity_block,
        grid=grid,
        in_specs=[
            pl.BlockSpec((_RPB, _N), lambda i: (i, 0)),
            pl.BlockSpec((_S, _N), lambda i: (0, 0)),
            pl.BlockSpec((_S, _RPB), lambda i: (0, i)),
        ],
        out_specs=[
            pl.BlockSpec((_RPB, _N), lambda i: (i, 0)),
            pl.BlockSpec((1, 1), lambda i: (0, 0)),
            pl.BlockSpec((1, 1), lambda i: (0, 0)),
            pl.BlockSpec((1, 1), lambda i: (0, 0)),
        ],
        out_shape=[
            jax.ShapeDtypeStruct((_N, _N), jnp.float32),
            jax.ShapeDtypeStruct((1, 1), jnp.int32),
            jax.ShapeDtypeStruct((1, 1), jnp.int32),
            jax.ShapeDtypeStruct((1, 1), jnp.float32),
        ],
    )(weight, activations, activations)
    return w_out, created[0, 0], pruned[0, 0], spars[0, 0]


# speedup vs baseline: 108.6063x; 1.2162x over previous
import jax
import jax.numpy as jnp
from jax.experimental import pallas as pl

_CREATE_THRESHOLD = 0.8
_PRUNE_THRESHOLD = 0.01
_MAX_SYNAPSES = 100
_MIN_SYNAPSES = 10
_INIT_STRENGTH = 0.01

_N = 4096
_S = 512
_RPB = 256


def _plasticity_block(w_ref, act_ref, actc_ref, out_ref,
                      created_ref, pruned_ref, spars_ref):
    i = pl.program_id(0)

    @pl.when(i == 0)
    def _init():
        created_ref[...] = jnp.zeros_like(created_ref)
        pruned_ref[...] = jnp.zeros_like(pruned_ref)
        spars_ref[...] = jnp.zeros_like(spars_ref)

    w = w_ref[...]
    absw = jnp.abs(w)
    strong = absw >= _PRUNE_THRESHOLD
    p_strong = jnp.where(strong, 1.0, 0.0)
    ones = jnp.ones((_N, 1), jnp.float32)
    cnt_strong = jax.lax.dot_general(
        p_strong, ones, (((1,), (0,)), ((), ())),
        preferred_element_type=jnp.float32)
    fast = jnp.all(cnt_strong >= float(_MAX_SYNAPSES))

    @pl.when(fast)
    def _fast_path():
        out_ref[...] = w * p_strong
        n_weak = float(_N * _RPB) - jnp.sum(cnt_strong)
        pruned_ref[...] += n_weak.astype(jnp.int32).reshape(1, 1)
        spars_ref[...] += n_weak.reshape(1, 1)

    @pl.when(jnp.logical_not(fast))
    def _general_path():
        cnt = jnp.sum((absw > 1e-10).astype(jnp.int32), axis=1, keepdims=True)
        has_room = cnt < _MAX_SYNAPSES

        def _prune_and_tally(wc, created_add):
            a = jnp.abs(wc)
            weak = a < _PRUNE_THRESHOLD
            count2 = jnp.sum((a >= _PRUNE_THRESHOLD).astype(jnp.int32),
                             axis=1, keepdims=True)
            pm = weak & (count2 > _MIN_SYNAPSES)
            w2 = jnp.where(pm, 0.0, wc)
            out_ref[...] = w2
            created_ref[...] += created_add.reshape(1, 1)
            pruned_ref[...] += jnp.sum(pm.astype(jnp.int32)).reshape(1, 1)
            spars_ref[...] += jnp.sum(
                (jnp.abs(w2) < 1e-10).astype(jnp.float32)).reshape(1, 1)

        @pl.when(jnp.any(has_room))
        def _rare_create_path():
            def _norm(x):
                mu = jnp.mean(x, axis=0, keepdims=True)
                c = x - mu
                var = jnp.sum(c * c, axis=0, keepdims=True) / (_S - 1)
                return c / jnp.maximum(jnp.sqrt(var), 1e-8)

            nfull = _norm(act_ref[...])
            ncol = _norm(actc_ref[...])
            corr = jax.lax.dot_general(
                ncol, nfull, (((0,), (0,)), ((), ())),
                preferred_element_type=jnp.float32) * (1.0 / _S)

            abscorr = jnp.abs(corr)
            cand = (abscorr > _CREATE_THRESHOLD) & (absw < 1e-10)
            cand_cnt = jnp.sum(cand.astype(jnp.int32), axis=1, keepdims=True)
            room = jnp.maximum(_MAX_SYNAPSES - cnt, 0)
            n = jnp.where(has_room, jnp.minimum(cand_cnt, room), 0)

            m = jnp.where(cand, abscorr, 0.0)
            mb = jax.lax.bitcast_convert_type(m, jnp.int32)

            def _bis_val(_, carry):
                lo, hi = carry
                mid = lo + (hi - lo) // 2
                c_ge = jnp.sum((mb >= mid).astype(jnp.int32), axis=1,
                               keepdims=True)
                ge = c_ge >= n
                return jnp.where(ge, mid, lo), jnp.where(ge, hi, mid)

            lo0 = jnp.zeros((_RPB, 1), jnp.int32)
            hi0 = jnp.full((_RPB, 1), 0x7F800000, jnp.int32)
            t, _ = jax.lax.fori_loop(0, 31, _bis_val, (lo0, hi0))

            strict = jnp.sum((mb > t).astype(jnp.int32), axis=1,
                             keepdims=True)
            need_eq = n - strict
            eq = (mb == t) & cand
            col = jax.lax.broadcasted_iota(jnp.int32, (_RPB, _N), 1)

            def _bis_col(_, carry):
                lo2, hi2 = carry
                mid = lo2 + (hi2 - lo2) // 2
                c_le = jnp.sum((eq & (col <= mid)).astype(jnp.int32), axis=1,
                               keepdims=True)
                ok = c_le >= need_eq
                return jnp.where(ok, lo2, mid), jnp.where(ok, mid, hi2)

            lo2 = jnp.full((_RPB, 1), -1, jnp.int32)
            hi2 = jnp.full((_RPB, 1), _N - 1, jnp.int32)
            _, cidx = jax.lax.fori_loop(0, 13, _bis_col, (lo2, hi2))

            create = ((mb > t) | (eq & (col <= cidx))) & cand & (n > 0)
            wc = jnp.where(create, _INIT_STRENGTH * jnp.sign(corr), w)
            _prune_and_tally(wc, jnp.sum(n))

        @pl.when(jnp.logical_not(jnp.any(has_room)))
        def _no_create_path():
            _prune_and_tally(w, jnp.int32(0))

    @pl.when(i == pl.num_programs(0) - 1)
    def _finish():
        spars_ref[...] = spars_ref[...] * (1.0 / (_N * _N))


def kernel(weight, activations):
    w_out, created, pruned, spars = pl.pallas_call(
        _plasticity_block,
        grid=(_N // _RPB,),
        in_specs=[
            pl.BlockSpec((_RPB, _N), lambda i: (i, 0)),
            pl.BlockSpec((_S, _N), lambda i: (0, 0)),
            pl.BlockSpec((_S, _RPB), lambda i: (0, i)),
        ],
        out_specs=[
            pl.BlockSpec((_RPB, _N), lambda i: (i, 0)),
            pl.BlockSpec((1, 1), lambda i: (0, 0)),
            pl.BlockSpec((1, 1), lambda i: (0, 0)),
            pl.BlockSpec((1, 1), lambda i: (0, 0)),
        ],
        out_shape=[
            jax.ShapeDtypeStruct((_N, _N), jnp.float32),
            jax.ShapeDtypeStruct((1, 1), jnp.int32),
            jax.ShapeDtypeStruct((1, 1), jnp.int32),
            jax.ShapeDtypeStruct((1, 1), jnp.float32),
        ],
    )(weight, activations, activations)
    return w_out, created[0, 0], pruned[0, 0], spars[0, 0]


# speedup vs baseline: 121.2940x; 1.1168x over previous
import jax
import jax.numpy as jnp
from jax.experimental import pallas as pl
from jax.experimental.pallas import tpu as pltpu

_CREATE_THRESHOLD = 0.8
_PRUNE_THRESHOLD = 0.01
_MAX_SYNAPSES = 100
_MIN_SYNAPSES = 10
_INIT_STRENGTH = 0.01

_N = 4096
_S = 512
_RPB = 256


def _plasticity_block(w_ref, act_hbm, out_ref,
                      created_ref, pruned_ref, spars_ref,
                      act_vmem, dma_sem):
    i = pl.program_id(0)

    @pl.when(i == 0)
    def _init():
        created_ref[...] = jnp.zeros_like(created_ref)
        pruned_ref[...] = jnp.zeros_like(pruned_ref)
        spars_ref[...] = jnp.zeros_like(spars_ref)

    w = w_ref[...]
    absw = jnp.abs(w)
    strong = absw >= _PRUNE_THRESHOLD
    p_strong = jnp.where(strong, 1.0, 0.0)
    ones = jnp.ones((_N, 1), jnp.float32)
    cnt_strong = jax.lax.dot_general(
        p_strong, ones, (((1,), (0,)), ((), ())),
        preferred_element_type=jnp.float32)
    fast = jnp.all(cnt_strong >= float(_MAX_SYNAPSES))

    @pl.when(fast)
    def _fast_path():
        out_ref[...] = w * p_strong
        n_weak = float(_N * _RPB) - jnp.sum(cnt_strong)
        pruned_ref[...] += n_weak.astype(jnp.int32).reshape(1, 1)
        spars_ref[...] += n_weak.reshape(1, 1)

    @pl.when(jnp.logical_not(fast))
    def _general_path():
        cnt = jnp.sum((absw > 1e-10).astype(jnp.int32), axis=1, keepdims=True)
        has_room = cnt < _MAX_SYNAPSES

        def _prune_and_tally(wc, created_add):
            a = jnp.abs(wc)
            weak = a < _PRUNE_THRESHOLD
            count2 = jnp.sum((a >= _PRUNE_THRESHOLD).astype(jnp.int32),
                             axis=1, keepdims=True)
            pm = weak & (count2 > _MIN_SYNAPSES)
            w2 = jnp.where(pm, 0.0, wc)
            out_ref[...] = w2
            created_ref[...] += created_add.reshape(1, 1)
            pruned_ref[...] += jnp.sum(pm.astype(jnp.int32)).reshape(1, 1)
            spars_ref[...] += jnp.sum(
                (jnp.abs(w2) < 1e-10).astype(jnp.float32)).reshape(1, 1)

        @pl.when(jnp.any(has_room))
        def _rare_create_path():
            cp = pltpu.make_async_copy(act_hbm, act_vmem, dma_sem)
            cp.start()
            cp.wait()

            def _norm(x):
                mu = jnp.mean(x, axis=0, keepdims=True)
                c = x - mu
                var = jnp.sum(c * c, axis=0, keepdims=True) / (_S - 1)
                return c / jnp.maximum(jnp.sqrt(var), 1e-8)

            nfull = _norm(act_vmem[...])
            ncol = _norm(act_vmem[:, pl.ds(i * _RPB, _RPB)])
            corr = jax.lax.dot_general(
                ncol, nfull, (((0,), (0,)), ((), ())),
                preferred_element_type=jnp.float32) * (1.0 / _S)

            abscorr = jnp.abs(corr)
            cand = (abscorr > _CREATE_THRESHOLD) & (absw < 1e-10)
            cand_cnt = jnp.sum(cand.astype(jnp.int32), axis=1, keepdims=True)
            room = jnp.maximum(_MAX_SYNAPSES - cnt, 0)
            n = jnp.where(has_room, jnp.minimum(cand_cnt, room), 0)

            m = jnp.where(cand, abscorr, 0.0)
            mb = jax.lax.bitcast_convert_type(m, jnp.int32)

            def _bis_val(_, carry):
                lo, hi = carry
                mid = lo + (hi - lo) // 2
                c_ge = jnp.sum((mb >= mid).astype(jnp.int32), axis=1,
                               keepdims=True)
                ge = c_ge >= n
                return jnp.where(ge, mid, lo), jnp.where(ge, hi, mid)

            lo0 = jnp.zeros((_RPB, 1), jnp.int32)
            hi0 = jnp.full((_RPB, 1), 0x7F800000, jnp.int32)
            t, _ = jax.lax.fori_loop(0, 31, _bis_val, (lo0, hi0))

            strict = jnp.sum((mb > t).astype(jnp.int32), axis=1,
                             keepdims=True)
            need_eq = n - strict
            eq = (mb == t) & cand
            col = jax.lax.broadcasted_iota(jnp.int32, (_RPB, _N), 1)

            def _bis_col(_, carry):
                lo2, hi2 = carry
                mid = lo2 + (hi2 - lo2) // 2
                c_le = jnp.sum((eq & (col <= mid)).astype(jnp.int32), axis=1,
                               keepdims=True)
                ok = c_le >= need_eq
                return jnp.where(ok, lo2, mid), jnp.where(ok, mid, hi2)

            lo2 = jnp.full((_RPB, 1), -1, jnp.int32)
            hi2 = jnp.full((_RPB, 1), _N - 1, jnp.int32)
            _, cidx = jax.lax.fori_loop(0, 13, _bis_col, (lo2, hi2))

            create = ((mb > t) | (eq & (col <= cidx))) & cand & (n > 0)
            wc = jnp.where(create, _INIT_STRENGTH * jnp.sign(corr), w)
            _prune_and_tally(wc, jnp.sum(n))

        @pl.when(jnp.logical_not(jnp.any(has_room)))
        def _no_create_path():
            _prune_and_tally(w, jnp.int32(0))

    @pl.when(i == pl.num_programs(0) - 1)
    def _finish():
        spars_ref[...] = spars_ref[...] * (1.0 / (_N * _N))


def kernel(weight, activations):
    w_out, created, pruned, spars = pl.pallas_call(
        _plasticity_block,
        grid=(_N // _RPB,),
        in_specs=[
            pl.BlockSpec((_RPB, _N), lambda i: (i, 0)),
            pl.BlockSpec(memory_space=pltpu.MemorySpace.HBM),
        ],
        out_specs=[
            pl.BlockSpec((_RPB, _N), lambda i: (i, 0)),
            pl.BlockSpec((1, 1), lambda i: (0, 0)),
            pl.BlockSpec((1, 1), lambda i: (0, 0)),
            pl.BlockSpec((1, 1), lambda i: (0, 0)),
        ],
        out_shape=[
            jax.ShapeDtypeStruct((_N, _N), jnp.float32),
            jax.ShapeDtypeStruct((1, 1), jnp.int32),
            jax.ShapeDtypeStruct((1, 1), jnp.int32),
            jax.ShapeDtypeStruct((1, 1), jnp.float32),
        ],
        scratch_shapes=[
            pltpu.VMEM((_S, _N), jnp.float32),
            pltpu.SemaphoreType.DMA,
        ],
    )(weight, activations)
    return w_out, created[0, 0], pruned[0, 0], spars[0, 0]


# speedup vs baseline: 133.3978x; 1.0998x over previous
import jax
import jax.numpy as jnp
from jax.experimental import pallas as pl

_CREATE_THRESHOLD = 0.8
_PRUNE_THRESHOLD = 0.01
_MAX_SYNAPSES = 100
_MIN_SYNAPSES = 10
_INIT_STRENGTH = 0.01

_N = 4096
_S = 512
_RPB = 512
_RPB_G = 256


def _fast_block(w_ref, out_ref, pruned_ref, spars_ref, ok_ref):
    i = pl.program_id(0)

    @pl.when(i == 0)
    def _init():
        pruned_ref[...] = jnp.zeros_like(pruned_ref)
        spars_ref[...] = jnp.zeros_like(spars_ref)
        ok_ref[...] = jnp.ones_like(ok_ref)

    w = w_ref[...]
    strong = jnp.abs(w) >= _PRUNE_THRESHOLD
    p_strong = jnp.where(strong, 1.0, 0.0)
    ones = jnp.ones((_N, 1), jnp.float32)
    cnt_strong = jax.lax.dot_general(
        p_strong, ones, (((1,), (0,)), ((), ())),
        preferred_element_type=jnp.float32)
    ok_blk = jnp.all(cnt_strong >= float(_MAX_SYNAPSES))
    ok_ref[...] &= jnp.where(ok_blk, 1, 0).reshape(1, 1)
    out_ref[...] = w * p_strong
    n_weak = float(_N * _RPB) - jnp.sum(cnt_strong)
    pruned_ref[...] += n_weak.astype(jnp.int32).reshape(1, 1)
    spars_ref[...] += n_weak.reshape(1, 1)

    @pl.when(i == pl.num_programs(0) - 1)
    def _finish():
        spars_ref[...] = spars_ref[...] * (1.0 / (_N * _N))


def _general_block(w_ref, act_ref, actc_ref, out_ref,
                   created_ref, pruned_ref, spars_ref):
    i = pl.program_id(0)

    @pl.when(i == 0)
    def _init():
        created_ref[...] = jnp.zeros_like(created_ref)
        pruned_ref[...] = jnp.zeros_like(pruned_ref)
        spars_ref[...] = jnp.zeros_like(spars_ref)

    w = w_ref[...]
    absw = jnp.abs(w)
    cnt = jnp.sum((absw > 1e-10).astype(jnp.int32), axis=1, keepdims=True)
    has_room = cnt < _MAX_SYNAPSES

    def _prune_and_tally(wc, created_add):
        a = jnp.abs(wc)
        weak = a < _PRUNE_THRESHOLD
        count2 = jnp.sum((a >= _PRUNE_THRESHOLD).astype(jnp.int32),
                         axis=1, keepdims=True)
        pm = weak & (count2 > _MIN_SYNAPSES)
        w2 = jnp.where(pm, 0.0, wc)
        out_ref[...] = w2
        created_ref[...] += created_add.reshape(1, 1)
        pruned_ref[...] += jnp.sum(pm.astype(jnp.int32)).reshape(1, 1)
        spars_ref[...] += jnp.sum(
            (jnp.abs(w2) < 1e-10).astype(jnp.float32)).reshape(1, 1)

    @pl.when(jnp.any(has_room))
    def _create_path():
        def _norm(x):
            mu = jnp.mean(x, axis=0, keepdims=True)
            c = x - mu
            var = jnp.sum(c * c, axis=0, keepdims=True) / (_S - 1)
            return c / jnp.maximum(jnp.sqrt(var), 1e-8)

        nfull = _norm(act_ref[...])
        ncol = _norm(actc_ref[...])
        corr = jax.lax.dot_general(
            ncol, nfull, (((0,), (0,)), ((), ())),
            preferred_element_type=jnp.float32) * (1.0 / _S)

        abscorr = jnp.abs(corr)
        cand = (abscorr > _CREATE_THRESHOLD) & (absw < 1e-10)
        cand_cnt = jnp.sum(cand.astype(jnp.int32), axis=1, keepdims=True)
        room = jnp.maximum(_MAX_SYNAPSES - cnt, 0)
        n = jnp.where(has_room, jnp.minimum(cand_cnt, room), 0)

        m = jnp.where(cand, abscorr, 0.0)
        mb = jax.lax.bitcast_convert_type(m, jnp.int32)

        def _bis_val(_, carry):
            lo, hi = carry
            mid = lo + (hi - lo) // 2
            c_ge = jnp.sum((mb >= mid).astype(jnp.int32), axis=1,
                           keepdims=True)
            ge = c_ge >= n
            return jnp.where(ge, mid, lo), jnp.where(ge, hi, mid)

        lo0 = jnp.zeros((_RPB_G, 1), jnp.int32)
        hi0 = jnp.full((_RPB_G, 1), 0x7F800000, jnp.int32)
        t, _ = jax.lax.fori_loop(0, 31, _bis_val, (lo0, hi0))

        strict = jnp.sum((mb > t).astype(jnp.int32), axis=1, keepdims=True)
        need_eq = n - strict
        eq = (mb == t) & cand
        col = jax.lax.broadcasted_iota(jnp.int32, (_RPB_G, _N), 1)

        def _bis_col(_, carry):
            lo2, hi2 = carry
            mid = lo2 + (hi2 - lo2) // 2
            c_le = jnp.sum((eq & (col <= mid)).astype(jnp.int32), axis=1,
                           keepdims=True)
            ok = c_le >= need_eq
            return jnp.where(ok, lo2, mid), jnp.where(ok, mid, hi2)

        lo2 = jnp.full((_RPB_G, 1), -1, jnp.int32)
        hi2 = jnp.full((_RPB_G, 1), _N - 1, jnp.int32)
        _, cidx = jax.lax.fori_loop(0, 13, _bis_col, (lo2, hi2))

        create = ((mb > t) | (eq & (col <= cidx))) & cand & (n > 0)
        wc = jnp.where(create, _INIT_STRENGTH * jnp.sign(corr), w)
        _prune_and_tally(wc, jnp.sum(n))

    @pl.when(jnp.logical_not(jnp.any(has_room)))
    def _no_create_path():
        _prune_and_tally(w, jnp.int32(0))

    @pl.when(i == pl.num_programs(0) - 1)
    def _finish():
        spars_ref[...] = spars_ref[...] * (1.0 / (_N * _N))


def _scalar_spec():
    return pl.BlockSpec((1, 1), lambda i: (0, 0))


def _run_general(weight, activations):
    w_out, created, pruned, spars = pl.pallas_call(
        _general_block,
        grid=(_N // _RPB_G,),
        in_specs=[
            pl.BlockSpec((_RPB_G, _N), lambda i: (i, 0)),
            pl.BlockSpec((_S, _N), lambda i: (0, 0)),
            pl.BlockSpec((_S, _RPB_G), lambda i: (0, i)),
        ],
        out_specs=[
            pl.BlockSpec((_RPB_G, _N), lambda i: (i, 0)),
            _scalar_spec(), _scalar_spec(), _scalar_spec(),
        ],
        out_shape=[
            jax.ShapeDtypeStruct((_N, _N), jnp.float32),
            jax.ShapeDtypeStruct((1, 1), jnp.int32),
            jax.ShapeDtypeStruct((1, 1), jnp.int32),
            jax.ShapeDtypeStruct((1, 1), jnp.float32),
        ],
    )(weight, activations, activations)
    return w_out, created[0, 0], pruned[0, 0], spars[0, 0]


def kernel(weight, activations):
    w_fast, pruned_f, spars_f, ok = pl.pallas_call(
        _fast_block,
        grid=(_N // _RPB,),
        in_specs=[pl.BlockSpec((_RPB, _N), lambda i: (i, 0))],
        out_specs=[
            pl.BlockSpec((_RPB, _N), lambda i: (i, 0)),
            _scalar_spec(), _scalar_spec(), _scalar_spec(),
        ],
        out_shape=[
            jax.ShapeDtypeStruct((_N, _N), jnp.float32),
            jax.ShapeDtypeStruct((1, 1), jnp.int32),
            jax.ShapeDtypeStruct((1, 1), jnp.float32),
            jax.ShapeDtypeStruct((1, 1), jnp.int32),
        ],
    )(weight)

    return jax.lax.cond(
        ok[0, 0] > 0,
        lambda w, a: (w_fast, jnp.int32(0), pruned_f[0, 0], spars_f[0, 0]),
        _run_general,
        weight, activations)
